# Initial kernel scaffold; baseline (speedup 1.0000x reference)
import functools
import jax
import jax.numpy as jnp
from jax import lax
from jax.experimental import pallas as pl
from jax.experimental.pallas import tpu as pltpu
from jax.experimental.pallas import tpu_sc as plsc

LANE = 16
CHUNK = 256
SUB = 128
EPS = 1e-12


def _vrsqrt(x):
    i = plsc.bitcast(x, jnp.int32)
    y = plsc.bitcast(jnp.int32(0x5F3759DF) - lax.shift_right_arithmetic(i, 1),
                     jnp.float32)
    for _ in range(3):
        y = y * (1.5 - 0.5 * x * y * y)
    return y


@functools.lru_cache(maxsize=None)
def _make_sc_kernel(n_rows, dim, n_combo):
    info = plsc.get_sparse_core_info()
    n_workers = info.num_cores * info.num_subcores
    assert n_rows % (n_workers * CHUNK) == 0
    rows_per_w = n_rows // n_workers
    n_chunks = rows_per_w // CHUNK
    kblk = dim // LANE
    idx_rows = CHUNK // SUB

    @functools.partial(
        pl.kernel,
        out_type=jax.ShapeDtypeStruct((n_rows, dim), jnp.float32),
        mesh=plsc.VectorSubcoreMesh(core_axis_name="c", subcore_axis_name="s"),
        scratch_types=[
            pltpu.VMEM((idx_rows, SUB), jnp.int32),
            pltpu.VMEM((idx_rows, SUB), jnp.int32),
            pltpu.VMEM((CHUNK, dim), jnp.float32),
            pltpu.VMEM((n_combo, dim), jnp.float32),
            pltpu.VMEM((2, dim), jnp.float32),
            pltpu.VMEM((2 * LANE,), jnp.float32),
            pltpu.SemaphoreType.DMA,
        ],
    )
    def sc_kernel(xf_hbm, cidx_hbm, tok_hbm, combo_hbm, gam_hbm, bet_hbm,
                  out_hbm, idx_v, cid_v, rows_v, combo_v, gb_v, ur_v, sem):
        wid = lax.axis_index("s") * info.num_cores + lax.axis_index("c")
        pltpu.sync_copy(combo_hbm, combo_v)
        pltpu.sync_copy(gam_hbm, gb_v.at[0])
        pltpu.sync_copy(bet_hbm, gb_v.at[1])
        gvec = [gb_v[0, pl.ds(k * LANE, LANE)] for k in range(kblk)]
        bvec = [gb_v[1, pl.ds(k * LANE, LANE)] for k in range(kblk)]

        def chunk_body(t, carry):
            base = wid * rows_per_w + t * CHUNK
            r0 = base // SUB
            pltpu.sync_copy(xf_hbm.at[pl.ds(r0, idx_rows)], idx_v)
            pltpu.sync_copy(cidx_hbm.at[pl.ds(r0, idx_rows)], cid_v)
            cps = [
                pltpu.async_copy(tok_hbm.at[idx_v.at[i]],
                                 rows_v.at[pl.ds(i * SUB, SUB)], sem)
                for i in range(idx_rows)
            ]
            for cp in cps:
                cp.wait()

            def group_body(g, c1):
                row16 = g * LANE + lax.iota(jnp.int32, LANE)
                cidx16 = cid_v[g // (SUB // LANE),
                               pl.ds((g % (SUB // LANE)) * LANE, LANE)]
                acc_s = jnp.zeros((LANE,), jnp.float32)
                acc_q = jnp.zeros((LANE,), jnp.float32)
                for d in range(dim):
                    spl = jnp.full((LANE,), d, jnp.int32)
                    tv = plsc.load_gather(rows_v, [row16, spl])
                    cv = plsc.load_gather(combo_v, [cidx16, spl])
                    e = tv + cv
                    acc_s = acc_s + e
                    acc_q = acc_q + e * e
                    plsc.store_scatter(rows_v, [row16, spl], e)
                u = acc_s * (1.0 / dim)
                var = acc_q * (1.0 / dim) - u * u
                r = _vrsqrt(var + EPS)
                ur_v[pl.ds(0, LANE)] = u
                ur_v[pl.ds(LANE, LANE)] = r

                def row_body(jj, c2):
                    j = g * LANE + jj
                    us = plsc.load_gather(ur_v, [jnp.full((LANE,), jj, jnp.int32)])
                    rs = plsc.load_gather(
                        ur_v, [jnp.full((LANE,), jj + LANE, jnp.int32)])
                    for k in range(kblk):
                        e_k = rows_v[j, pl.ds(k * LANE, LANE)]
                        rows_v[j, pl.ds(k * LANE, LANE)] = (
                            gvec[k] * ((e_k - us) * rs) + bvec[k])
                    return c2

                lax.fori_loop(0, LANE, row_body, 0)
                return c1

            lax.fori_loop(0, CHUNK // LANE, group_body, 0)
            pltpu.sync_copy(rows_v, out_hbm.at[pl.ds(base, CHUNK)])
            return carry

        lax.fori_loop(0, n_chunks, chunk_body, 0)

    return sc_kernel


def kernel(x, seg, tok_table, pos_table, seg_table, gamma, beta):
    b, l = x.shape
    vocab, dim = tok_table.shape
    n_seg = seg_table.shape[0]
    n = b * l
    xf = x.astype(jnp.int32).reshape(n // SUB, SUB)
    pos_ids = jnp.arange(l, dtype=jnp.int32)[None, :]
    cidx = (pos_ids * n_seg + seg.astype(jnp.int32)).reshape(n // SUB, SUB)
    combo = (pos_table[:l, None, :] + seg_table[None, :, :]).reshape(
        l * n_seg, dim)
    out = _make_sc_kernel(n, dim, l * n_seg)(
        xf, cidx, tok_table, combo, gamma, beta)
    return out.reshape(b, l, dim)


# speedup vs baseline: 5.2371x; 5.2371x over previous
import functools
import jax
import jax.numpy as jnp
from jax import lax
from jax.experimental import pallas as pl
from jax.experimental.pallas import tpu as pltpu
from jax.experimental.pallas import tpu_sc as plsc

LANE = 16
CHUNK = 256
SUB = 128
EPS = 1e-12
_TAKE_DNUMS = lax.GatherDimensionNumbers(
    offset_dims=(), collapsed_slice_dims=(0,), start_index_map=(0,))


def _lane_take(v, idx):
    return lax.gather(v, idx[:, None], dimension_numbers=_TAKE_DNUMS,
                      slice_sizes=(1,),
                      mode=lax.GatherScatterMode.PROMISE_IN_BOUNDS)


def _vrsqrt(x):
    i = lax.bitcast_convert_type(x, jnp.int32)
    y = lax.bitcast_convert_type(
        jnp.int32(0x5F3759DF) - lax.shift_right_arithmetic(i, 1), jnp.float32)
    for _ in range(3):
        y = y * (1.5 - 0.5 * x * y * y)
    return y


def _lane_butterfly_sum(v):
    for step in (1, 2, 4, 8):
        perm = jnp.arange(LANE, dtype=jnp.int32) ^ step
        v = v + _lane_take(v, perm)
    return v


@functools.lru_cache(maxsize=None)
def _make_sc_kernel(n_rows, dim, n_pos):
    info = plsc.get_sparse_core_info()
    n_workers = info.num_cores * info.num_subcores
    assert n_rows % (n_workers * CHUNK) == 0
    rows_per_w = n_rows // n_workers
    n_chunks = rows_per_w // CHUNK
    kblk = dim // LANE

    @functools.partial(
        pl.kernel,
        out_type=jax.ShapeDtypeStruct((n_rows, dim), jnp.float32),
        mesh=plsc.VectorSubcoreMesh(core_axis_name="c", subcore_axis_name="s"),
        scratch_types=[
            pltpu.VMEM((CHUNK,), jnp.int32),
            pltpu.VMEM((CHUNK,), jnp.int32),
            pltpu.VMEM((CHUNK, dim), jnp.float32),
            pltpu.VMEM((n_pos, dim), jnp.float32),
            pltpu.VMEM((2, dim), jnp.float32),
            pltpu.VMEM((2, dim), jnp.float32),
            pltpu.SemaphoreType.DMA,
        ],
    )
    def sc_kernel(xf_hbm, segf_hbm, tok_hbm, pos_hbm, segtab_hbm, gam_hbm,
                  bet_hbm, out_hbm, idx_v, seg_v, rows_v, pos_v, st_v, gb_v,
                  sem):
        wid = lax.axis_index("s") * info.num_cores + lax.axis_index("c")
        pltpu.sync_copy(pos_hbm, pos_v)
        pltpu.sync_copy(segtab_hbm, st_v)
        pltpu.sync_copy(gam_hbm, gb_v.at[0])
        pltpu.sync_copy(bet_hbm, gb_v.at[1])
        gvec = [gb_v[0, pl.ds(k * LANE, LANE)] for k in range(kblk)]
        bvec = [gb_v[1, pl.ds(k * LANE, LANE)] for k in range(kblk)]
        seg0 = [st_v[0, pl.ds(k * LANE, LANE)] for k in range(kblk)]
        sdif = [st_v[1, pl.ds(k * LANE, LANE)] - seg0[k] for k in range(kblk)]

        def chunk_body(t, carry):
            base = wid * rows_per_w + t * CHUNK
            pltpu.sync_copy(xf_hbm.at[pl.ds(base, CHUNK)], idx_v)
            pltpu.sync_copy(segf_hbm.at[pl.ds(base, CHUNK)], seg_v)
            cps = [
                pltpu.async_copy(tok_hbm.at[idx_v.at[pl.ds(i * SUB, SUB)]],
                                 rows_v.at[pl.ds(i * SUB, SUB)], sem)
                for i in range(CHUNK // SUB)
            ]
            for cp in cps:
                cp.wait()

            def group_body(g, c1):
                seg16 = seg_v[pl.ds(g * LANE, LANE)]
                pbase = base + g * LANE

                def row_body(jj, c2):
                    j = g * LANE + jj
                    p = lax.rem(pbase + jj, n_pos)
                    s_spl = _lane_take(seg16, jnp.full((LANE,), jj, jnp.int32))
                    sf = s_spl.astype(jnp.float32)
                    e = []
                    for k in range(kblk):
                        tok_k = rows_v[j, pl.ds(k * LANE, LANE)]
                        pos_k = pos_v[p, pl.ds(k * LANE, LANE)]
                        e.append(tok_k + pos_k + (seg0[k] + sf * sdif[k]))
                    acc = e[0]
                    accq = e[0] * e[0]
                    for k in range(1, kblk):
                        acc = acc + e[k]
                        accq = accq + e[k] * e[k]
                    usum = _lane_butterfly_sum(acc)
                    qsum = _lane_butterfly_sum(accq)
                    u = usum * (1.0 / dim)
                    var = qsum * (1.0 / dim) - u * u
                    r = _vrsqrt(var + EPS)
                    for k in range(kblk):
                        rows_v[j, pl.ds(k * LANE, LANE)] = (
                            gvec[k] * ((e[k] - u) * r) + bvec[k])
                    return c2

                lax.fori_loop(0, LANE, row_body, 0)
                return c1

            lax.fori_loop(0, CHUNK // LANE, group_body, 0)
            pltpu.sync_copy(rows_v, out_hbm.at[pl.ds(base, CHUNK)])
            return carry

        lax.fori_loop(0, n_chunks, chunk_body, 0)

    return sc_kernel


def kernel(x, seg, tok_table, pos_table, seg_table, gamma, beta):
    b, l = x.shape
    vocab, dim = tok_table.shape
    n = b * l
    xf = x.astype(jnp.int32).reshape(n)
    segf = seg.astype(jnp.int32).reshape(n)
    out = _make_sc_kernel(n, dim, l)(
        xf, segf, tok_table, pos_table[:l], seg_table, gamma, beta)
    return out.reshape(b, l, dim)


# speedup vs baseline: 5.4765x; 1.0457x over previous
import functools
import jax
import jax.numpy as jnp
from jax import lax
from jax.experimental import pallas as pl
from jax.experimental.pallas import tpu as pltpu
from jax.experimental.pallas import tpu_sc as plsc

LANE = 16
CHUNK = 256
SUB = 128
EPS = 1e-12
_TAKE_DNUMS = lax.GatherDimensionNumbers(
    offset_dims=(), collapsed_slice_dims=(0,), start_index_map=(0,))


def _lane_take(v, idx):
    return lax.gather(v, idx[:, None], dimension_numbers=_TAKE_DNUMS,
                      slice_sizes=(1,),
                      mode=lax.GatherScatterMode.PROMISE_IN_BOUNDS)


def _vrsqrt(x):
    i = lax.bitcast_convert_type(x, jnp.int32)
    y = lax.bitcast_convert_type(
        jnp.int32(0x5F3759DF) - lax.shift_right_arithmetic(i, 1), jnp.float32)
    for _ in range(2):
        y = y * (1.5 - 0.5 * x * y * y)
    return y


def _lane_butterfly_sum(v):
    for step in (1, 2, 4, 8):
        perm = jnp.arange(LANE, dtype=jnp.int32) ^ step
        v = v + _lane_take(v, perm)
    return v


@functools.lru_cache(maxsize=None)
def _make_sc_kernel(n_rows, dim, n_pos):
    info = plsc.get_sparse_core_info()
    n_workers = info.num_cores * info.num_subcores
    assert n_rows % (n_workers * CHUNK) == 0
    rows_per_w = n_rows // n_workers
    n_chunks = rows_per_w // CHUNK
    kblk = dim // LANE

    @functools.partial(
        pl.kernel,
        out_type=jax.ShapeDtypeStruct((n_rows, dim), jnp.float32),
        mesh=plsc.VectorSubcoreMesh(core_axis_name="c", subcore_axis_name="s"),
        scratch_types=[
            pltpu.VMEM((CHUNK,), jnp.int32),
            pltpu.VMEM((CHUNK,), jnp.int32),
            pltpu.VMEM((CHUNK, dim), jnp.float32),
            pltpu.VMEM((n_pos, dim), jnp.float32),
            pltpu.VMEM((2, dim), jnp.float32),
            pltpu.VMEM((2, dim), jnp.float32),
            pltpu.SemaphoreType.DMA,
        ],
    )
    def sc_kernel(xf_hbm, segf_hbm, tok_hbm, pos_hbm, segtab_hbm, gam_hbm,
                  bet_hbm, out_hbm, idx_v, seg_v, rows_v, pos_v, st_v, gb_v,
                  sem):
        wid = lax.axis_index("s") * info.num_cores + lax.axis_index("c")
        pltpu.sync_copy(pos_hbm, pos_v)
        pltpu.sync_copy(segtab_hbm, st_v)
        pltpu.sync_copy(gam_hbm, gb_v.at[0])
        pltpu.sync_copy(bet_hbm, gb_v.at[1])
        gvec = [gb_v[0, pl.ds(k * LANE, LANE)] for k in range(kblk)]
        bvec = [gb_v[1, pl.ds(k * LANE, LANE)] for k in range(kblk)]
        seg0 = [st_v[0, pl.ds(k * LANE, LANE)] for k in range(kblk)]
        sdif = [st_v[1, pl.ds(k * LANE, LANE)] - seg0[k] for k in range(kblk)]

        def chunk_body(t, carry):
            base = wid * rows_per_w + t * CHUNK
            pltpu.sync_copy(xf_hbm.at[pl.ds(base, CHUNK)], idx_v)
            pltpu.sync_copy(segf_hbm.at[pl.ds(base, CHUNK)], seg_v)
            cps = [
                pltpu.async_copy(tok_hbm.at[idx_v.at[pl.ds(i * SUB, SUB)]],
                                 rows_v.at[pl.ds(i * SUB, SUB)], sem)
                for i in range(CHUNK // SUB)
            ]
            for cp in cps:
                cp.wait()

            def group_body(g, c1):
                seg16 = seg_v[pl.ds(g * LANE, LANE)]
                pbase = base + g * LANE

                def one_row(jj):
                    j = g * LANE + jj
                    p = lax.rem(pbase + jj, n_pos)
                    s_spl = _lane_take(seg16, jnp.full((LANE,), jj, jnp.int32))
                    sf = s_spl.astype(jnp.float32)
                    e = []
                    for k in range(kblk):
                        tok_k = rows_v[j, pl.ds(k * LANE, LANE)]
                        pos_k = pos_v[p, pl.ds(k * LANE, LANE)]
                        e.append(tok_k + pos_k + (seg0[k] + sf * sdif[k]))
                    acc = e[0]
                    accq = e[0] * e[0]
                    for k in range(1, kblk):
                        acc = acc + e[k]
                        accq = accq + e[k] * e[k]
                    usum = _lane_butterfly_sum(acc)
                    qsum = _lane_butterfly_sum(accq)
                    u = usum * (1.0 / dim)
                    var = qsum * (1.0 / dim) - u * u
                    r = _vrsqrt(var + EPS)
                    for k in range(kblk):
                        rows_v[j, pl.ds(k * LANE, LANE)] = (
                            gvec[k] * ((e[k] - u) * r) + bvec[k])

                def row_body(jj2, c2):
                    one_row(jj2 * 2)
                    one_row(jj2 * 2 + 1)
                    return c2

                lax.fori_loop(0, LANE // 2, row_body, 0)
                return c1

            lax.fori_loop(0, CHUNK // LANE, group_body, 0)
            pltpu.sync_copy(rows_v, out_hbm.at[pl.ds(base, CHUNK)])
            return carry

        lax.fori_loop(0, n_chunks, chunk_body, 0)

    return sc_kernel


def kernel(x, seg, tok_table, pos_table, seg_table, gamma, beta):
    b, l = x.shape
    vocab, dim = tok_table.shape
    n = b * l
    xf = x.astype(jnp.int32).reshape(n)
    segf = seg.astype(jnp.int32).reshape(n)
    out = _make_sc_kernel(n, dim, l)(
        xf, segf, tok_table, pos_table[:l], seg_table, gamma, beta)
    return out.reshape(b, l, dim)


# speedup vs baseline: 6.9776x; 1.2741x over previous
import functools
import jax
import jax.numpy as jnp
from jax import lax
from jax.experimental import pallas as pl
from jax.experimental.pallas import tpu as pltpu
from jax.experimental.pallas import tpu_sc as plsc

LANE = 16
CHUNK = 256
SUB = 128
NBUF = 3
EPS = 1e-12
_TAKE_DNUMS = lax.GatherDimensionNumbers(
    offset_dims=(), collapsed_slice_dims=(0,), start_index_map=(0,))


def _lane_take(v, idx):
    return lax.gather(v, idx[:, None], dimension_numbers=_TAKE_DNUMS,
                      slice_sizes=(1,),
                      mode=lax.GatherScatterMode.PROMISE_IN_BOUNDS)


def _vrsqrt(x):
    i = lax.bitcast_convert_type(x, jnp.int32)
    y = lax.bitcast_convert_type(
        jnp.int32(0x5F3759DF) - lax.shift_right_arithmetic(i, 1), jnp.float32)
    for _ in range(2):
        y = y * (1.5 - 0.5 * x * y * y)
    return y


def _lane_butterfly_sum(v):
    for step in (1, 2, 4, 8):
        perm = jnp.arange(LANE, dtype=jnp.int32) ^ step
        v = v + _lane_take(v, perm)
    return v


@functools.lru_cache(maxsize=None)
def _make_sc_kernel(n_rows, dim, n_pos):
    info = plsc.get_sparse_core_info()
    n_workers = info.num_cores * info.num_subcores
    assert n_rows % (n_workers * CHUNK) == 0
    rows_per_w = n_rows // n_workers
    n_chunks = rows_per_w // CHUNK
    kblk = dim // LANE
    n_sub = CHUNK // SUB

    @functools.partial(
        pl.kernel,
        out_type=jax.ShapeDtypeStruct((n_rows, dim), jnp.float32),
        mesh=plsc.VectorSubcoreMesh(core_axis_name="c", subcore_axis_name="s"),
        scratch_types=[
            pltpu.VMEM((2, CHUNK), jnp.int32),
            pltpu.VMEM((2, CHUNK), jnp.int32),
            pltpu.VMEM((NBUF, CHUNK, dim), jnp.float32),
            pltpu.VMEM((n_pos, dim), jnp.float32),
            pltpu.VMEM((3, dim), jnp.float32),
            pltpu.SemaphoreType.DMA,
            pltpu.SemaphoreType.DMA,
        ],
    )
    def sc_kernel(xf_hbm, segf_hbm, tok_hbm, pos0_hbm, sdif_hbm, gam_hbm,
                  bet_hbm, out_hbm, idx_v, seg_v, rows_v, pos_v, gbs_v,
                  sem_g, sem_o):
        wid = lax.axis_index("s") * info.num_cores + lax.axis_index("c")
        wbase = wid * rows_per_w
        pltpu.sync_copy(pos0_hbm, pos_v)
        pltpu.sync_copy(gam_hbm, gbs_v.at[0])
        pltpu.sync_copy(bet_hbm, gbs_v.at[1])
        pltpu.sync_copy(sdif_hbm, gbs_v.at[2])
        gvec = [gbs_v[0, pl.ds(k * LANE, LANE)] for k in range(kblk)]
        bvec = [gbs_v[1, pl.ds(k * LANE, LANE)] for k in range(kblk)]
        sdif = [gbs_v[2, pl.ds(k * LANE, LANE)] for k in range(kblk)]

        def issue_gather(ib, rb):
            for i in range(n_sub):
                pltpu.async_copy(
                    tok_hbm.at[idx_v.at[ib, pl.ds(i * SUB, SUB)]],
                    rows_v.at[rb, pl.ds(i * SUB, SUB)], sem_g)

        def drain_gather(ib, rb):
            for i in range(n_sub):
                pltpu.make_async_copy(
                    tok_hbm.at[idx_v.at[ib, pl.ds(i * SUB, SUB)]],
                    rows_v.at[rb, pl.ds(i * SUB, SUB)], sem_g).wait()

        def drain_out(rb, base):
            pltpu.make_async_copy(
                rows_v.at[rb], out_hbm.at[pl.ds(base, CHUNK)], sem_o).wait()

        pltpu.sync_copy(xf_hbm.at[pl.ds(wbase, CHUNK)], idx_v.at[0])
        pltpu.sync_copy(segf_hbm.at[pl.ds(wbase, CHUNK)], seg_v.at[0])
        issue_gather(0, 0)

        def chunk_body(t, carry):
            b = lax.rem(t, NBUF)
            bn = lax.rem(t + 1, NBUF)
            ib = lax.rem(t, 2)
            ibn = lax.rem(t + 1, 2)
            base = wbase + t * CHUNK
            tn = jnp.minimum(t + 1, n_chunks - 1)
            basen = wbase + tn * CHUNK

            @pl.when(t >= 2)
            def _():
                drain_out(bn, wbase + (t - 2) * CHUNK)

            pltpu.sync_copy(xf_hbm.at[pl.ds(basen, CHUNK)], idx_v.at[ibn])
            pltpu.sync_copy(segf_hbm.at[pl.ds(basen, CHUNK)], seg_v.at[ibn])
            issue_gather(ibn, bn)

            drain_gather(ib, b)

            def group_body(g, c1):
                seg16 = seg_v[ib, pl.ds(g * LANE, LANE)]
                pbase = base + g * LANE

                def one_row(jj):
                    j = g * LANE + jj
                    p = lax.rem(pbase + jj, n_pos)
                    s_spl = _lane_take(seg16, jnp.full((LANE,), jj, jnp.int32))
                    sf = s_spl.astype(jnp.float32)
                    e = []
                    for k in range(kblk):
                        tok_k = rows_v[b, j, pl.ds(k * LANE, LANE)]
                        pos_k = pos_v[p, pl.ds(k * LANE, LANE)]
                        e.append((tok_k + pos_k) + sf * sdif[k])
                    acc = e[0]
                    accq = e[0] * e[0]
                    for k in range(1, kblk):
                        acc = acc + e[k]
                        accq = accq + e[k] * e[k]
                    usum = _lane_butterfly_sum(acc)
                    qsum = _lane_butterfly_sum(accq)
                    u = usum * (1.0 / dim)
                    var = qsum * (1.0 / dim) - u * u
                    r = _vrsqrt(var + EPS)
                    for k in range(kblk):
                        rows_v[b, j, pl.ds(k * LANE, LANE)] = (
                            gvec[k] * ((e[k] - u) * r) + bvec[k])

                def row_body(q, c2):
                    for u4 in range(4):
                        one_row(q * 4 + u4)
                    return c2

                lax.fori_loop(0, LANE // 4, row_body, 0)
                return c1

            lax.fori_loop(0, CHUNK // LANE, group_body, 0)

            pltpu.async_copy(rows_v.at[b], out_hbm.at[pl.ds(base, CHUNK)],
                             sem_o)
            return carry

        lax.fori_loop(0, n_chunks, chunk_body, 0)

        drain_gather(lax.rem(n_chunks, 2), lax.rem(n_chunks, NBUF))
        drain_out(lax.rem(n_chunks - 2, NBUF), wbase + (n_chunks - 2) * CHUNK)
        drain_out(lax.rem(n_chunks - 1, NBUF), wbase + (n_chunks - 1) * CHUNK)

    return sc_kernel


def kernel(x, seg, tok_table, pos_table, seg_table, gamma, beta):
    b, l = x.shape
    vocab, dim = tok_table.shape
    n = b * l
    xf = x.astype(jnp.int32).reshape(n)
    segf = seg.astype(jnp.int32).reshape(n)
    pos0 = pos_table[:l] + seg_table[0]
    sdif = seg_table[1] - seg_table[0]
    out = _make_sc_kernel(n, dim, l)(
        xf, segf, tok_table, pos0, sdif, gamma, beta)
    return out.reshape(b, l, dim)


# speedup vs baseline: 8.5534x; 1.2258x over previous
import functools
import jax
import jax.numpy as jnp
from jax import lax
from jax.experimental import pallas as pl
from jax.experimental.pallas import tpu as pltpu
from jax.experimental.pallas import tpu_sc as plsc

LANE = 16
CHUNK = 256
SUB = 128
NBUF = 3
EPS = 1e-12
_TAKE_DNUMS = lax.GatherDimensionNumbers(
    offset_dims=(), collapsed_slice_dims=(0,), start_index_map=(0,))


def _lane_take(v, idx):
    return lax.gather(v, idx[:, None], dimension_numbers=_TAKE_DNUMS,
                      slice_sizes=(1,),
                      mode=lax.GatherScatterMode.PROMISE_IN_BOUNDS)


def _vrsqrt(x):
    i = lax.bitcast_convert_type(x, jnp.int32)
    y = lax.bitcast_convert_type(
        jnp.int32(0x5F3759DF) - lax.shift_right_arithmetic(i, 1), jnp.float32)
    for _ in range(2):
        y = y * (1.5 - 0.5 * x * y * y)
    return y


def _lane_butterfly_sum(v):
    for step in (1, 2, 4, 8):
        perm = jnp.arange(LANE, dtype=jnp.int32) ^ step
        v = v + _lane_take(v, perm)
    return v


@functools.lru_cache(maxsize=None)
def _make_sc_kernel(n_rows, dim, n_pos):
    info = plsc.get_sparse_core_info()
    n_workers = info.num_cores * info.num_subcores
    assert n_rows % (n_workers * CHUNK) == 0
    rows_per_w = n_rows // n_workers
    n_chunks = rows_per_w // CHUNK
    kblk = dim // LANE
    n_sub = CHUNK // SUB

    @functools.partial(
        pl.kernel,
        out_type=jax.ShapeDtypeStruct((n_rows, dim), jnp.float32),
        mesh=plsc.VectorSubcoreMesh(core_axis_name="c", subcore_axis_name="s"),
        scratch_types=[
            pltpu.VMEM((2, CHUNK), jnp.int32),
            pltpu.VMEM((2, CHUNK), jnp.int32),
            pltpu.VMEM((NBUF, CHUNK, dim), jnp.float32),
            pltpu.VMEM((n_pos, dim), jnp.float32),
            pltpu.VMEM((3, dim), jnp.float32),
            pltpu.SemaphoreType.DMA,
            pltpu.SemaphoreType.DMA,
        ],
    )
    def sc_kernel(xf_hbm, segf_hbm, tok_hbm, pos0_hbm, sdif_hbm, gam_hbm,
                  bet_hbm, out_hbm, idx_v, seg_v, rows_v, pos_v, gbs_v,
                  sem_g, sem_o):
        wid = lax.axis_index("s") * info.num_cores + lax.axis_index("c")
        wbase = wid * rows_per_w
        pltpu.sync_copy(pos0_hbm, pos_v)
        pltpu.sync_copy(gam_hbm, gbs_v.at[0])
        pltpu.sync_copy(bet_hbm, gbs_v.at[1])
        pltpu.sync_copy(sdif_hbm, gbs_v.at[2])
        gvec = [gbs_v[0, pl.ds(k * LANE, LANE)] for k in range(kblk)]
        bvec = [gbs_v[1, pl.ds(k * LANE, LANE)] for k in range(kblk)]
        sdif = [gbs_v[2, pl.ds(k * LANE, LANE)] for k in range(kblk)]

        def issue_gather(ib, rb):
            for i in range(n_sub):
                pltpu.async_copy(
                    tok_hbm.at[idx_v.at[ib, pl.ds(i * SUB, SUB)]],
                    rows_v.at[rb, pl.ds(i * SUB, SUB)], sem_g)

        def drain_gather(ib, rb):
            for i in range(n_sub):
                pltpu.make_async_copy(
                    tok_hbm.at[idx_v.at[ib, pl.ds(i * SUB, SUB)]],
                    rows_v.at[rb, pl.ds(i * SUB, SUB)], sem_g).wait()

        def drain_out(rb, base):
            pltpu.make_async_copy(
                rows_v.at[rb], out_hbm.at[pl.ds(base, CHUNK)], sem_o).wait()

        pltpu.sync_copy(xf_hbm.at[pl.ds(wbase, CHUNK)], idx_v.at[0])
        pltpu.sync_copy(segf_hbm.at[pl.ds(wbase, CHUNK)], seg_v.at[0])
        issue_gather(0, 0)

        def chunk_body(t, carry):
            b = lax.rem(t, NBUF)
            bn = lax.rem(t + 1, NBUF)
            ib = lax.rem(t, 2)
            ibn = lax.rem(t + 1, 2)
            base = wbase + t * CHUNK
            tn = jnp.minimum(t + 1, n_chunks - 1)
            basen = wbase + tn * CHUNK

            @pl.when(t >= 2)
            def _():
                drain_out(bn, wbase + (t - 2) * CHUNK)

            pltpu.sync_copy(xf_hbm.at[pl.ds(basen, CHUNK)], idx_v.at[ibn])
            pltpu.sync_copy(segf_hbm.at[pl.ds(basen, CHUNK)], seg_v.at[ibn])
            issue_gather(ibn, bn)

            drain_gather(ib, b)

            def group_body(g, c1):
                seg16 = seg_v[ib, pl.ds(g * LANE, LANE)]
                pbase = base + g * LANE

                def one_row(jj):
                    j = g * LANE + jj
                    p = lax.rem(pbase + jj, n_pos)
                    s_spl = _lane_take(seg16, jnp.full((LANE,), jj, jnp.int32))
                    sf = s_spl.astype(jnp.float32)
                    e = []
                    for k in range(kblk):
                        tok_k = rows_v[b, j, pl.ds(k * LANE, LANE)]
                        pos_k = pos_v[p, pl.ds(k * LANE, LANE)]
                        e.append((tok_k + pos_k) + sf * sdif[k])
                    acc = e[0]
                    accq = e[0] * e[0]
                    for k in range(1, kblk):
                        acc = acc + e[k]
                        accq = accq + e[k] * e[k]
                    usum = _lane_butterfly_sum(acc)
                    qsum = _lane_butterfly_sum(accq)
                    u = usum * (1.0 / dim)
                    var = qsum * (1.0 / dim) - u * u
                    r = _vrsqrt(var + EPS)
                    for k in range(kblk):
                        rows_v[b, j, pl.ds(k * LANE, LANE)] = (
                            gvec[k] * ((e[k] - u) * r) + bvec[k])

                @plsc.parallel_loop(0, LANE, 1, unroll=4)
                def _row(jj):
                    one_row(jj)

                return c1

            lax.fori_loop(0, CHUNK // LANE, group_body, 0)

            pltpu.async_copy(rows_v.at[b], out_hbm.at[pl.ds(base, CHUNK)],
                             sem_o)
            return carry

        lax.fori_loop(0, n_chunks, chunk_body, 0)

        drain_gather(lax.rem(n_chunks, 2), lax.rem(n_chunks, NBUF))
        drain_out(lax.rem(n_chunks - 2, NBUF), wbase + (n_chunks - 2) * CHUNK)
        drain_out(lax.rem(n_chunks - 1, NBUF), wbase + (n_chunks - 1) * CHUNK)

    return sc_kernel


def kernel(x, seg, tok_table, pos_table, seg_table, gamma, beta):
    b, l = x.shape
    vocab, dim = tok_table.shape
    n = b * l
    xf = x.astype(jnp.int32).reshape(n)
    segf = seg.astype(jnp.int32).reshape(n)
    pos0 = pos_table[:l] + seg_table[0]
    sdif = seg_table[1] - seg_table[0]
    out = _make_sc_kernel(n, dim, l)(
        xf, segf, tok_table, pos0, sdif, gamma, beta)
    return out.reshape(b, l, dim)


# speedup vs baseline: 9.9219x; 1.1600x over previous
import functools
import jax
import jax.numpy as jnp
from jax import lax
from jax.experimental import pallas as pl
from jax.experimental.pallas import tpu as pltpu
from jax.experimental.pallas import tpu_sc as plsc

LANE = 16
CHUNK = 256
SUB = 128
NBUF = 3
EPS = 1e-12
_TAKE_DNUMS = lax.GatherDimensionNumbers(
    offset_dims=(), collapsed_slice_dims=(0,), start_index_map=(0,))


def _lane_take(v, idx):
    return lax.gather(v, idx[:, None], dimension_numbers=_TAKE_DNUMS,
                      slice_sizes=(1,),
                      mode=lax.GatherScatterMode.PROMISE_IN_BOUNDS)


def _vrsqrt(x):
    i = lax.bitcast_convert_type(x, jnp.int32)
    y = lax.bitcast_convert_type(
        jnp.int32(0x5F3759DF) - lax.shift_right_arithmetic(i, 1), jnp.float32)
    for _ in range(2):
        y = y * (1.5 - 0.5 * x * y * y)
    return y


def _lane_butterfly_sum(v):
    for step in (1, 2, 4, 8):
        perm = jnp.arange(LANE, dtype=jnp.int32) ^ step
        v = v + _lane_take(v, perm)
    return v


@functools.lru_cache(maxsize=None)
def _make_sc_kernel(n_rows, dim, n_pos):
    info = plsc.get_sparse_core_info()
    n_workers = info.num_cores * info.num_subcores
    assert n_rows % (n_workers * CHUNK) == 0
    rows_per_w = n_rows // n_workers
    n_chunks = rows_per_w // CHUNK
    kblk = dim // LANE
    n_sub = CHUNK // SUB

    @functools.partial(
        pl.kernel,
        out_type=jax.ShapeDtypeStruct((n_rows, dim), jnp.float32),
        mesh=plsc.VectorSubcoreMesh(core_axis_name="c", subcore_axis_name="s"),
        scratch_types=[
            pltpu.VMEM((2, CHUNK), jnp.int32),
            pltpu.VMEM((2, CHUNK), jnp.int32),
            pltpu.VMEM((NBUF, CHUNK, dim), jnp.float32),
            pltpu.VMEM((n_pos, dim), jnp.float32),
            pltpu.VMEM((3, dim), jnp.float32),
            pltpu.SemaphoreType.DMA,
            pltpu.SemaphoreType.DMA,
        ],
    )
    def sc_kernel(xf_hbm, segf_hbm, tok_hbm, pos0_hbm, sdif_hbm, gam_hbm,
                  bet_hbm, out_hbm, idx_v, seg_v, rows_v, pos_v, gbs_v,
                  sem_g, sem_o):
        wid = lax.axis_index("s") * info.num_cores + lax.axis_index("c")
        wbase = wid * rows_per_w
        pltpu.sync_copy(pos0_hbm, pos_v)
        pltpu.sync_copy(gam_hbm, gbs_v.at[0])
        pltpu.sync_copy(bet_hbm, gbs_v.at[1])
        pltpu.sync_copy(sdif_hbm, gbs_v.at[2])
        gvec = [gbs_v[0, pl.ds(k * LANE, LANE)] for k in range(kblk)]
        bvec = [gbs_v[1, pl.ds(k * LANE, LANE)] for k in range(kblk)]
        sdif = [gbs_v[2, pl.ds(k * LANE, LANE)] for k in range(kblk)]

        def issue_gather(ib, rb):
            for i in range(n_sub):
                pltpu.async_copy(
                    tok_hbm.at[idx_v.at[ib, pl.ds(i * SUB, SUB)]],
                    rows_v.at[rb, pl.ds(i * SUB, SUB)], sem_g)

        def drain_gather(ib, rb):
            for i in range(n_sub):
                pltpu.make_async_copy(
                    tok_hbm.at[idx_v.at[ib, pl.ds(i * SUB, SUB)]],
                    rows_v.at[rb, pl.ds(i * SUB, SUB)], sem_g).wait()

        def drain_out(rb, base):
            pltpu.make_async_copy(
                rows_v.at[rb], out_hbm.at[pl.ds(base, CHUNK)], sem_o).wait()

        pltpu.sync_copy(xf_hbm.at[pl.ds(wbase, CHUNK)], idx_v.at[0])
        pltpu.sync_copy(segf_hbm.at[pl.ds(wbase, CHUNK)], seg_v.at[0])
        issue_gather(0, 0)

        def chunk_body(t, carry):
            b = lax.rem(t, NBUF)
            bn = lax.rem(t + 1, NBUF)
            ib = lax.rem(t, 2)
            ibn = lax.rem(t + 1, 2)
            base = wbase + t * CHUNK
            tn = jnp.minimum(t + 1, n_chunks - 1)
            basen = wbase + tn * CHUNK

            @pl.when(t >= 2)
            def _():
                drain_out(bn, wbase + (t - 2) * CHUNK)

            pltpu.sync_copy(xf_hbm.at[pl.ds(basen, CHUNK)], idx_v.at[ibn])
            pltpu.sync_copy(segf_hbm.at[pl.ds(basen, CHUNK)], seg_v.at[ibn])
            issue_gather(ibn, bn)

            drain_gather(ib, b)

            def group_body(g, c1):
                seg16 = seg_v[ib, pl.ds(g * LANE, LANE)]
                pbase = base + g * LANE

                def one_row(jj):
                    j = g * LANE + jj
                    p = lax.rem(pbase + jj, n_pos)
                    s_spl = _lane_take(seg16, jnp.full((LANE,), jj, jnp.int32))
                    sf = s_spl.astype(jnp.float32)
                    e = []
                    for k in range(kblk):
                        tok_k = rows_v[b, j, pl.ds(k * LANE, LANE)]
                        pos_k = pos_v[p, pl.ds(k * LANE, LANE)]
                        e.append((tok_k + pos_k) + sf * sdif[k])
                    acc = e[0]
                    accq = e[0] * e[0]
                    for k in range(1, kblk):
                        acc = acc + e[k]
                        accq = accq + e[k] * e[k]
                    usum = _lane_butterfly_sum(acc)
                    qsum = _lane_butterfly_sum(accq)
                    u = usum * (1.0 / dim)
                    var = qsum * (1.0 / dim) - u * u
                    r = _vrsqrt(var + EPS)
                    for k in range(kblk):
                        rows_v[b, j, pl.ds(k * LANE, LANE)] = (
                            gvec[k] * ((e[k] - u) * r) + bvec[k])

                @plsc.parallel_loop(0, LANE, 1, unroll=8)
                def _row(jj):
                    one_row(jj)

                return c1

            lax.fori_loop(0, CHUNK // LANE, group_body, 0)

            pltpu.async_copy(rows_v.at[b], out_hbm.at[pl.ds(base, CHUNK)],
                             sem_o)
            return carry

        lax.fori_loop(0, n_chunks, chunk_body, 0)

        drain_gather(lax.rem(n_chunks, 2), lax.rem(n_chunks, NBUF))
        drain_out(lax.rem(n_chunks - 2, NBUF), wbase + (n_chunks - 2) * CHUNK)
        drain_out(lax.rem(n_chunks - 1, NBUF), wbase + (n_chunks - 1) * CHUNK)

    return sc_kernel


def kernel(x, seg, tok_table, pos_table, seg_table, gamma, beta):
    b, l = x.shape
    vocab, dim = tok_table.shape
    n = b * l
    xf = x.astype(jnp.int32).reshape(n)
    segf = seg.astype(jnp.int32).reshape(n)
    pos0 = pos_table[:l] + seg_table[0]
    sdif = seg_table[1] - seg_table[0]
    out = _make_sc_kernel(n, dim, l)(
        xf, segf, tok_table, pos0, sdif, gamma, beta)
    return out.reshape(b, l, dim)
